# Initial kernel scaffold; baseline (speedup 1.0000x reference)
import jax
import jax.numpy as jnp
from jax import lax
from jax.experimental import pallas as pl
from jax.experimental.pallas import tpu as pltpu
from jax.experimental.pallas import tpu_sc as plsc

_N = 10000
_E = 320000
_D = 128
_H = 4
_DH = 32

_NC = 2
_NS = 16
_NW = _NC * _NS

_KVW = 272
_AGW = 144

_EW = _E // _NW
_C = 80
_NCHUNK = _EW // _C
_RPT = _N // _NS
_ZB = 125


def _proj_body(x_ref, wt_ref, ws_ref, wc_ref, q_ref, kv_ref):
    xb = x_ref[...]
    q_ref[...] = jax.lax.dot_general(
        xb, wt_ref[...], (((1,), (1,)), ((), ())),
        preferred_element_type=jnp.float32)
    k = jax.lax.dot_general(
        xb, ws_ref[...], (((1,), (1,)), ((), ())),
        preferred_element_type=jnp.float32)
    v = jax.lax.dot_general(
        xb, wc_ref[...], (((1,), (1,)), ((), ())),
        preferred_element_type=jnp.float32)
    b = xb.shape[0]
    kv_ref[:, 0:_D] = k
    kv_ref[:, _D:2 * _D] = v
    kv_ref[:, 2 * _D:_KVW] = jnp.where(
        lax.broadcasted_iota(jnp.int32, (b, _KVW - 2 * _D), 1) < _H, 1.0, 0.0)


def _project(x, Wt, Ws, Wc):
    blk = 1000
    grid = (_N // blk,)
    return pl.pallas_call(
        _proj_body,
        grid=grid,
        in_specs=[
            pl.BlockSpec((blk, _D), lambda i: (i, 0)),
            pl.BlockSpec((_D, _D), lambda i: (0, 0)),
            pl.BlockSpec((_D, _D), lambda i: (0, 0)),
            pl.BlockSpec((_D, _D), lambda i: (0, 0)),
        ],
        out_specs=[
            pl.BlockSpec((blk, _D), lambda i: (i, 0)),
            pl.BlockSpec((blk, _KVW), lambda i: (i, 0)),
        ],
        out_shape=[
            jax.ShapeDtypeStruct((_N, _D), jnp.float32),
            jax.ShapeDtypeStruct((_N, _KVW), jnp.float32),
        ],
    )(x, Wt, Ws, Wc)


def _edge_body(qtab, kvtab, src, dst, aggout, src_v, dst_v, qb, kvb, msg, zb):
    cid = lax.axis_index("c")
    sid = lax.axis_index("s")
    wid = sid * _NC + cid

    def scoped(shared):
        iota = lax.iota(jnp.int32, 16)

        @pl.loop(0, _ZB)
        def _zero_zb(i):
            for j in range(_AGW // 16):
                zb[i, pl.ds(16 * j, 16)] = jnp.zeros((16,), jnp.float32)

        for j in range(_RPT // _ZB):
            pltpu.sync_copy(zb, shared.at[pl.ds(sid * _RPT + j * _ZB, _ZB)])
        plsc.subcore_barrier()

        @pl.loop(0, _NCHUNK)
        def _chunk(g):
            base = wid * _EW + g * _C
            pltpu.sync_copy(src.at[pl.ds(base, _C)], src_v)
            pltpu.sync_copy(dst.at[pl.ds(base, _C)], dst_v.at[0])
            pltpu.sync_copy(qtab.at[dst_v.at[0]], qb)
            pltpu.sync_copy(kvtab.at[src_v], kvb)

            @pl.loop(0, _C)
            def _edge(i):
                evs = []
                for h in range(_H):
                    p = (qb[i, pl.ds(32 * h, 16)] * kvb[i, pl.ds(32 * h, 16)]
                         + qb[i, pl.ds(32 * h + 16, 16)]
                         * kvb[i, pl.ds(32 * h + 16, 16)])
                    sh = jnp.sum(p)
                    ev = jnp.exp(jnp.broadcast_to(sh, (16,)))
                    evs.append(ev)
                    msg[i, pl.ds(32 * h, 16)] = (
                        ev * kvb[i, pl.ds(_D + 32 * h, 16)])
                    msg[i, pl.ds(32 * h + 16, 16)] = (
                        ev * kvb[i, pl.ds(_D + 32 * h + 16, 16)])
                evec = jnp.where(iota == 0, evs[0],
                                 jnp.where(iota == 1, evs[1],
                                           jnp.where(iota == 2, evs[2],
                                                     evs[3])))
                msg[i, pl.ds(_D, 16)] = evec * kvb[i, pl.ds(2 * _D, 16)]

            pltpu.sync_copy(msg, shared.at[dst_v.at[0]], add=True)

        plsc.subcore_barrier()

        for j in range(_RPT // _ZB):
            r0 = sid * _RPT + j * _ZB
            pltpu.sync_copy(shared.at[pl.ds(r0, _ZB)],
                            aggout.at[cid, pl.ds(r0, _ZB)])

    pl.run_scoped(scoped, shared=pltpu.VMEM_SHARED((_N, _AGW), jnp.float32))


def _edge_pass(qtab, kvtab, src, dst):
    mesh = plsc.VectorSubcoreMesh(core_axis_name="c", subcore_axis_name="s")
    return pl.kernel(
        _edge_body,
        out_type=jax.ShapeDtypeStruct((_NC, _N, _AGW), jnp.float32),
        mesh=mesh,
        scratch_types=[
            pltpu.VMEM((_C,), jnp.int32),
            pltpu.VMEM((1, _C), jnp.int32),
            pltpu.VMEM((_C, _D), jnp.float32),
            pltpu.VMEM((_C, _KVW), jnp.float32),
            pltpu.VMEM((_C, _AGW), jnp.float32),
            pltpu.VMEM((_ZB, _AGW), jnp.float32),
        ],
    )(qtab, kvtab, src, dst)


def _out_body(agg_ref, x_ref, wq_ref, bq_ref, o_ref):
    agg = agg_ref[0] + agg_ref[1]
    d = agg[:, _D:_D + _H]
    r2 = lax.broadcasted_iota(jnp.int32, (_H, _D), 1) // _DH
    r1 = lax.broadcasted_iota(jnp.int32, (_H, _D), 0)
    rmat = jnp.where(r1 == r2, 1.0, 0.0)
    drep = jax.lax.dot_general(d, rmat, (((1,), (0,)), ((), ())),
                               preferred_element_type=jnp.float32)
    num = agg[:, 0:_D] / (drep + 1e-16)
    out = jax.lax.dot_general(num, wq_ref[...], (((1,), (1,)), ((), ())),
                              preferred_element_type=jnp.float32)
    out = out + bq_ref[...]
    o_ref[...] = jnp.maximum(out, 0.0) + x_ref[...]


def _finish(agg, x, Wq, bq):
    blk = 1000
    grid = (_N // blk,)
    return pl.pallas_call(
        _out_body,
        grid=grid,
        in_specs=[
            pl.BlockSpec((_NC, blk, _AGW), lambda i: (0, i, 0)),
            pl.BlockSpec((blk, _D), lambda i: (i, 0)),
            pl.BlockSpec((_D, _D), lambda i: (0, 0)),
            pl.BlockSpec((1, _D), lambda i: (0, 0)),
        ],
        out_specs=pl.BlockSpec((blk, _D), lambda i: (i, 0)),
        out_shape=jax.ShapeDtypeStruct((_N, _D), jnp.float32),
    )(agg, x, Wq, bq)


def kernel(x, edge_index, Wt, Ws, Wc, Wq, bq):
    ei = edge_index.astype(jnp.int32)
    src = ei[0]
    dst = ei[1]
    qtab, kvtab = _project(x, Wt, Ws, Wc)
    agg = _edge_pass(qtab, kvtab, src, dst)
    return _finish(agg, x, Wq, bq.reshape(1, _D))


# speedup vs baseline: 13.2442x; 13.2442x over previous
import jax
import jax.numpy as jnp
from jax import lax
from jax.experimental import pallas as pl
from jax.experimental.pallas import tpu as pltpu
from jax.experimental.pallas import tpu_sc as plsc

_N = 10000
_E = 320000
_D = 128
_H = 4
_DH = 32

_NC = 2
_NS = 16
_NW = _NC * _NS

_EW = _E // _NW
_C = 80
_NG = _C // 16
_NCHUNK = _EW // _C
_C2 = 16
_NCHUNK2 = _EW // _C2
_NP = 10240
_RPT = _NP // _NS
_DR = 320
_ER = 640
_EB = 40


def _proj_body(x_ref, wt_ref, ws_ref, wc_ref, wq_ref, q_ref, k_ref, vp_ref):
    xb = x_ref[...]
    q_ref[...] = lax.dot_general(xb, wt_ref[...], (((1,), (1,)), ((), ())),
                                 preferred_element_type=jnp.float32)
    k_ref[...] = lax.dot_general(xb, ws_ref[...], (((1,), (1,)), ((), ())),
                                 preferred_element_type=jnp.float32)
    v = lax.dot_general(xb, wc_ref[...], (((1,), (1,)), ((), ())),
                        preferred_element_type=jnp.float32)
    wq = wq_ref[...]
    for h in range(_H):
        vh = v[:, h * _DH:(h + 1) * _DH]
        wqh = wq[:, h * _DH:(h + 1) * _DH]
        vp_ref[:, h, :] = lax.dot_general(
            vh, wqh, (((1,), (1,)), ((), ())),
            preferred_element_type=jnp.float32)


def _project(x, Wt, Ws, Wc, Wq):
    blk = 1000
    grid = (_N // blk,)
    return pl.pallas_call(
        _proj_body,
        grid=grid,
        in_specs=[
            pl.BlockSpec((blk, _D), lambda i: (i, 0)),
            pl.BlockSpec((_D, _D), lambda i: (0, 0)),
            pl.BlockSpec((_D, _D), lambda i: (0, 0)),
            pl.BlockSpec((_D, _D), lambda i: (0, 0)),
            pl.BlockSpec((_D, _D), lambda i: (0, 0)),
        ],
        out_specs=[
            pl.BlockSpec((blk, _D), lambda i: (i, 0)),
            pl.BlockSpec((blk, _D), lambda i: (i, 0)),
            pl.BlockSpec((blk, _H, _D), lambda i: (i, 0, 0)),
        ],
        out_shape=[
            jax.ShapeDtypeStruct((_N, _D), jnp.float32),
            jax.ShapeDtypeStruct((_N, _D), jnp.float32),
            jax.ShapeDtypeStruct((_N, _H, _D), jnp.float32),
        ],
    )(x, Wt, Ws, Wc, Wq)


def _p1_body(qtab, ktab, src, dst, ehbm, denout,
             src_v, dst_v, drow_v, qb, kb, dmsg, ebuf, den, sem):
    cid = lax.axis_index("c")
    sid = lax.axis_index("s")
    wid = sid * _NC + cid
    iota = lax.iota(jnp.int32, 16)
    zeros = jnp.zeros((16,), jnp.float32)

    @pl.loop(0, _C)
    def _zero_dmsg(i):
        for j in range(_D // 16):
            dmsg[i, pl.ds(16 * j, 16)] = zeros

    @pl.when(sid < _DR // _C)
    def _zero_den():
        pltpu.sync_copy(dmsg, den.at[pl.ds(sid * _C, _C)])

    plsc.subcore_barrier()

    @pl.loop(0, _NCHUNK)
    def _chunk(g):
        base = wid * _EW + g * _C
        pltpu.sync_copy(src.at[pl.ds(base, _C)], src_v)
        pltpu.sync_copy(dst.at[pl.ds(base, _C)], dst_v.at[0])
        cq = pltpu.async_copy(qtab.at[dst_v.at[0]], qb, sem)
        ck = pltpu.async_copy(ktab.at[src_v], kb, sem)
        cq.wait()
        ck.wait()

        for j in range(_NG):
            dv = dst_v[0, pl.ds(16 * j, 16)]
            drow_v[0, pl.ds(16 * j, 16)] = dv // 32

        @pl.loop(0, _C)
        def _edge(i):
            evs = []
            for h in range(_H):
                p = (qb[i, pl.ds(32 * h, 16)] * kb[i, pl.ds(32 * h, 16)]
                     + qb[i, pl.ds(32 * h + 16, 16)]
                     * kb[i, pl.ds(32 * h + 16, 16)])
                sh = jnp.sum(p)
                evs.append(jnp.exp(jnp.broadcast_to(sh, (16,))))
            evec = jnp.where(iota == 0, evs[0],
                             jnp.where(iota == 1, evs[1],
                                       jnp.where(iota == 2, evs[2],
                                                 evs[3])))
            g16 = (i // 16) * 16
            erow = (g % 8) * _NG + i // 16
            elane = (i - g16) * 4 + iota
            plsc.store_scatter(ebuf, [jnp.broadcast_to(erow, (16,)), elane],
                               evec, mask=iota < _H)
            dvec = dst_v[0, pl.ds(g16, 16)]
            di_vec = dvec[jnp.broadcast_to(i - g16, (16,))]
            lane = (di_vec % 32) * 4 + iota
            plsc.store_scatter(dmsg, [jnp.broadcast_to(i, (16,)), lane],
                               evec, mask=iota < _H)

        pltpu.sync_copy(dmsg, den.at[drow_v.at[0]], add=True)

        @pl.loop(0, _C)
        def _clear(i):
            g16 = (i // 16) * 16
            dvec = dst_v[0, pl.ds(g16, 16)]
            di_vec = dvec[jnp.broadcast_to(i - g16, (16,))]
            lane = (di_vec % 32) * 4 + iota
            plsc.store_scatter(dmsg, [jnp.broadcast_to(i, (16,)), lane],
                               zeros, mask=iota < _H)

        @pl.when(((g % 8) == 7) | (g == _NCHUNK - 1))
        def _flush():
            pltpu.sync_copy(ebuf, ehbm.at[wid, pl.ds((g // 8) * _EB, _EB)])

    plsc.subcore_barrier()

    @pl.when(sid < _DR // _C)
    def _out_den():
        pltpu.sync_copy(den.at[pl.ds(sid * _C, _C)],
                        denout.at[cid, pl.ds(sid * _C, _C)])


def _phase1(qtab, ktab, src, dst):
    mesh = plsc.VectorSubcoreMesh(core_axis_name="c", subcore_axis_name="s",
                                  num_cores=_NC, num_subcores=_NS)
    return pl.kernel(
        _p1_body,
        out_type=[
            jax.ShapeDtypeStruct((_NW, _ER, 64), jnp.float32),
            jax.ShapeDtypeStruct((_NC, _DR, _D), jnp.float32),
        ],
        mesh=mesh,
        compiler_params=pltpu.CompilerParams(needs_layout_passes=False),
        scratch_types=[
            pltpu.VMEM((_C,), jnp.int32),
            pltpu.VMEM((1, _C), jnp.int32),
            pltpu.VMEM((1, _C), jnp.int32),
            pltpu.VMEM((_C, _D), jnp.float32),
            pltpu.VMEM((_C, _D), jnp.float32),
            pltpu.VMEM((_C, _D), jnp.float32),
            pltpu.VMEM((_EB, 64), jnp.float32),
            pltpu.VMEM_SHARED((_DR, _D), jnp.float32),
            pltpu.SemaphoreType.DMA,
        ],
    )(qtab, ktab, src, dst)


def _p2_body(vptab, drec, src, dst, ehbm, aggout,
             src_v, dst_v, vpb, drb, msg, ebuf, agg, sem):
    cid = lax.axis_index("c")
    sid = lax.axis_index("s")
    wid = sid * _NC + cid
    iota = lax.iota(jnp.int32, 16)
    zeros = jnp.zeros((16,), jnp.float32)

    @pl.loop(0, _C2)
    def _zero_msg(i):
        for j in range(_D // 16):
            msg[i, pl.ds(16 * j, 16)] = zeros

    @pl.loop(0, _RPT // _C2)
    def _zero_agg(j):
        pltpu.sync_copy(msg, agg.at[pl.ds(sid * _RPT + j * _C2, _C2)])

    plsc.subcore_barrier()

    @pl.loop(0, _NCHUNK2)
    def _chunk(g):
        @pl.when((g % _EB) == 0)
        def _load_exp():
            pltpu.sync_copy(ehbm.at[wid, pl.ds((g // _EB) * _EB, _EB)], ebuf)

        base = wid * _EW + g * _C2
        pltpu.sync_copy(src.at[pl.ds(base, _C2)], src_v)
        pltpu.sync_copy(dst.at[pl.ds(base, _C2)], dst_v.at[0])
        cv = pltpu.async_copy(vptab.at[src_v], vpb, sem)
        cd = pltpu.async_copy(drec.at[dst_v.at[0]], drb, sem)
        cv.wait()
        cd.wait()

        @pl.loop(0, _C2)
        def _edge(i):
            erow = g % _EB
            elane = i * 4 + (iota & 3)
            e4 = plsc.load_gather(ebuf,
                                  [jnp.broadcast_to(erow, (16,)), elane])
            r4 = plsc.load_gather(drb, [jnp.broadcast_to(i, (16,)),
                                        (iota & 3) * _DH])
            a4 = e4 * r4
            ah = [a4[jnp.broadcast_to(h, (16,))] for h in range(_H)]
            for j in range(_D // 16):
                m = ah[0] * vpb[i, 0, pl.ds(16 * j, 16)]
                m = m + ah[1] * vpb[i, 1, pl.ds(16 * j, 16)]
                m = m + ah[2] * vpb[i, 2, pl.ds(16 * j, 16)]
                m = m + ah[3] * vpb[i, 3, pl.ds(16 * j, 16)]
                msg[i, pl.ds(16 * j, 16)] = m

        pltpu.sync_copy(msg, agg.at[dst_v.at[0]], add=True)

    plsc.subcore_barrier()

    pltpu.sync_copy(agg.at[pl.ds(sid * _RPT, _RPT)],
                    aggout.at[cid, pl.ds(sid * _RPT, _RPT)])


def _phase2(vptab, drec, src, dst, ehbm):
    mesh = plsc.VectorSubcoreMesh(core_axis_name="c", subcore_axis_name="s",
                                  num_cores=_NC, num_subcores=_NS)
    return pl.kernel(
        _p2_body,
        out_type=jax.ShapeDtypeStruct((_NC, _NP, _D), jnp.float32),
        mesh=mesh,
        compiler_params=pltpu.CompilerParams(needs_layout_passes=False),
        scratch_types=[
            pltpu.VMEM((_C2,), jnp.int32),
            pltpu.VMEM((1, _C2), jnp.int32),
            pltpu.VMEM((_C2, _H, _D), jnp.float32),
            pltpu.VMEM((_C2, _D), jnp.float32),
            pltpu.VMEM((_C2, _D), jnp.float32),
            pltpu.VMEM((_EB, 64), jnp.float32),
            pltpu.VMEM_SHARED((_NP, _D), jnp.float32),
            pltpu.SemaphoreType.DMA,
        ],
    )(vptab, drec, src, dst, ehbm)


def kernel(x, edge_index, Wt, Ws, Wc, Wq, bq):
    ei = edge_index.astype(jnp.int32)
    src = ei[0]
    dst = ei[1]
    qtab, ktab, vptab = _project(x, Wt, Ws, Wc, Wq)
    ehbm, den = _phase1(qtab, ktab, src, dst)
    dent = (den[0] + den[1]).reshape(_NP, _H)
    rec = 1.0 / (dent + 1e-16)
    drec = jnp.repeat(rec, _DH, axis=1)
    agg = _phase2(vptab, drec, src, dst, ehbm)
    pre = (agg[0] + agg[1])[:_N] + bq
    return jnp.maximum(pre, 0.0) + x
